# baseline (speedup 1.0000x reference)
import functools

import jax
import jax.numpy as jnp
import numpy as np
from jax.experimental import pallas as pl
from jax.experimental.pallas import tpu as pltpu

_N = 10000
_T = 21
_F_IN = 128
_HID = 32
_BN = 768
_SCALE = 1.0 / (1.0 + 1e-5) ** 0.5


def _perm_const(cin):
    p = np.zeros((3 * cin, 3 * cin), np.float32)
    for ci in range(cin):
        for k in range(3):
            p[ci * 3 + k, k * cin + ci] = 1.0
    return p


def _glu_t(Y):
    P = Y[0:32, :]
    Q = Y[32:64, :]
    R = Y[64:96, :]
    return jax.nn.relu(P * jax.nn.sigmoid(Q) + R)


def _tap_stack(H, t_out):
    L = t_out * _BN
    return jnp.concatenate(
        [H[:, 0:L], H[:, _BN:_BN + L], H[:, 2 * _BN:2 * _BN + L]], axis=0)


def _dot(a, b):
    return jnp.dot(a, b, preferred_element_type=jnp.float32)


def _bcol(b3):
    return jnp.transpose(jnp.concatenate(b3, axis=1), (1, 0))


def _stgcn_block(x_ref, mask_ref, p1_ref, p96_ref,
                 w1a_ref, w1b_ref, w1c_ref, b1a_ref, b1b_ref, b1c_ref,
                 w2a_ref, w2b_ref, w2c_ref, b2a_ref, b2b_ref, b2c_ref,
                 w3a_ref, w3b_ref, w3c_ref, b3a_ref, b3b_ref, b3c_ref,
                 w4a_ref, w4b_ref, w4c_ref, b4a_ref, b4b_ref, b4c_ref,
                 wca_ref, bca_ref, wcb_ref, bcb_ref, lw_ref, lb_ref,
                 out_ref, acc_ref, *, nblocks):
    i = pl.program_id(0)

    cat1 = jnp.concatenate(
        [w1a_ref[...], w1b_ref[...], w1c_ref[...]], axis=0).astype(jnp.bfloat16)
    W1 = _dot(cat1, p1_ref[...]).astype(jnp.bfloat16)
    B1 = _bcol([b1a_ref[...], b1b_ref[...], b1c_ref[...]])
    W2 = _dot(jnp.concatenate(
        [w2a_ref[...], w2b_ref[...], w2c_ref[...]], axis=0), p96_ref[...])
    B2 = _bcol([b2a_ref[...], b2b_ref[...], b2c_ref[...]])
    W3 = _dot(jnp.concatenate(
        [w3a_ref[...], w3b_ref[...], w3c_ref[...]], axis=0), p96_ref[...])
    B3 = _bcol([b3a_ref[...], b3b_ref[...], b3c_ref[...]])
    W4 = _dot(jnp.concatenate(
        [w4a_ref[...], w4b_ref[...], w4c_ref[...]], axis=0), p96_ref[...])
    B4 = _bcol([b4a_ref[...], b4b_ref[...], b4c_ref[...]])
    Bca = jnp.transpose(bca_ref[...], (1, 0))
    Bcb = jnp.transpose(bcb_ref[...], (1, 0))

    xb = x_ref[...].astype(jnp.bfloat16)
    X3 = jnp.transpose(xb, (0, 2, 1))
    xT = jnp.concatenate([X3[t] for t in range(_T)], axis=1)

    A0 = _dot(W1[:, 0:128], xT)
    A1 = _dot(W1[:, 128:256], xT)
    A2 = _dot(W1[:, 256:384], xT)
    L1 = 19 * _BN
    Y1 = A0[:, 0:L1] + A1[:, _BN:_BN + L1] + A2[:, 2 * _BN:2 * _BN + L1] + B1
    H1 = _glu_t(Y1)
    cheb = lambda w_ref, H: jax.lax.dot_general(
        w_ref[...], H, (((0,), (0,)), ((), ())),
        preferred_element_type=jnp.float32)
    Tc = jax.nn.relu(cheb(wca_ref, H1) + Bca)
    H2 = _glu_t(_dot(W2, _tap_stack(Tc, 17)) + B2) * _SCALE
    H3 = _glu_t(_dot(W3, _tap_stack(H2, 15)) + B3)
    Tc2 = jax.nn.relu(cheb(wcb_ref, H3) + Bcb)
    H4 = _glu_t(_dot(W4, _tap_stack(Tc2, 13)) + B4)

    mask = jnp.concatenate([mask_ref[0]] * 13, axis=1)
    H4 = jnp.where(mask > 0, H4, 0.0)
    part = jnp.sum(H4, axis=0, keepdims=True)

    @pl.when(i == 0)
    def _init():
        acc_ref[...] = jnp.zeros_like(acc_ref)

    acc_ref[...] += part

    @pl.when(i == nblocks - 1)
    def _finish():
        acc = acc_ref[...]
        a13 = jnp.concatenate(
            [acc[:, t * _BN:(t + 1) * _BN] for t in range(13)], axis=0)
        s = jnp.sum(a13, axis=1, keepdims=True)
        out = jnp.sum(s * lw_ref[...], axis=0, keepdims=True)
        out_ref[...] = out * (_SCALE / (_N * _HID)) + lb_ref[...]


def kernel(x, edge_index, edge_weight, tc1a, cheb_a, tc2a, tc1b, cheb_b, tc2b,
           lin_w, lin_b):
    del edge_index, edge_weight
    r2 = lambda w: w.reshape(_HID, -1)
    rb = lambda b: b.reshape(1, -1)
    nblocks = -(-_N // _BN)
    mask = np.arange(nblocks * _BN) < _N
    mask = jnp.asarray(mask.astype(np.float32).reshape(nblocks, 1, _BN))
    p1 = jnp.asarray(_perm_const(_F_IN).astype(np.float32))
    p96 = jnp.asarray(_perm_const(_HID))
    lb = lin_b.reshape(1, -1)

    operands = [x, mask, p1.astype(jnp.bfloat16), p96]
    for p in (tc1a, tc2a, tc1b, tc2b):
        operands += [r2(p[0]), r2(p[2]), r2(p[4])]
        operands += [rb(p[1]), rb(p[3]), rb(p[5])]
    operands += [cheb_a[0], rb(cheb_a[1]), cheb_b[0], rb(cheb_b[1]), lin_w, lb]

    full = lambda a: pl.BlockSpec(a.shape, lambda *_: tuple(0 for _ in a.shape))
    in_specs = [
        pl.BlockSpec((_T, _BN, _F_IN), lambda i: (0, i, 0)),
        pl.BlockSpec((1, 1, _BN), lambda i: (i, 0, 0)),
    ] + [full(a) for a in operands[2:]]
    out = pl.pallas_call(
        functools.partial(_stgcn_block, nblocks=nblocks),
        grid=(nblocks,),
        in_specs=in_specs,
        out_specs=pl.BlockSpec((1, lin_w.shape[1]), lambda i: (0, 0)),
        out_shape=jax.ShapeDtypeStruct((1, lin_w.shape[1]), jnp.float32),
        scratch_shapes=[pltpu.VMEM((1, 13 * _BN), jnp.float32)],
    )(*operands)
    return out[0]
